# baseline (speedup 1.0000x reference)
import functools

import jax
import jax.numpy as jnp
from jax import lax
from jax.experimental import pallas as pl
from jax.experimental.pallas import tpu as pltpu
from jax.experimental.pallas import tpu_sc as plsc

B, L, E, Mt, Mc, V = 4, 4, 1024, 1024, 512, 50000
F = 3 * E
GAMMA = 0.8
HALF = 24992
CB = 3200
NFULL = 7
TAIL = 2608
TW0 = 2592
BM = 512
BIGI = 2 ** 30


def _k1_body(h_ref, wt_ref, wg_ref, bg_ref, hp_ref, gate_ref):
    h = h_ref[...]
    hp_ref[...] = jnp.dot(h, wt_ref[...], preferred_element_type=jnp.float32)
    z = jnp.dot(h, wg_ref[...], preferred_element_type=jnp.float32)
    gate_ref[...] = jax.nn.sigmoid(z + bg_ref[0, 0])


def _k1(h16, W_triple, Wg_t, bg):
    return pl.pallas_call(
        _k1_body,
        out_shape=[jax.ShapeDtypeStruct((16, F), jnp.float32),
                   jax.ShapeDtypeStruct((16, 1), jnp.float32)],
        in_specs=[pl.BlockSpec((16, E), lambda: (0, 0)),
                  pl.BlockSpec((E, F), lambda: (0, 0)),
                  pl.BlockSpec((E, 1), lambda: (0, 0)),
                  pl.BlockSpec(memory_space=pltpu.SMEM)],
        out_specs=[pl.BlockSpec((16, F), lambda: (0, 0)),
                   pl.BlockSpec((16, 1), lambda: (0, 0))],
    )(h16, W_triple, Wg_t, bg)


def _k2_body(hp_ref, tr_ref, lab_ref, out_ref):
    hp = hp_ref[0]
    tr = tr_ref[0]
    logits = lax.dot_general(hp, tr, (((1,), (1,)), ((), ())),
                             preferred_element_type=jnp.float32)
    prob = jax.nn.sigmoid(logits)
    lab = lab_ref[0]
    out_ref[0] = jnp.where(lab == -1, 0.0, prob)


def _k2(hp, triple_repr, labels3):
    return pl.pallas_call(
        _k2_body,
        grid=(B, Mt // BM),
        out_shape=jax.ShapeDtypeStruct((B, L, Mt), jnp.float32),
        in_specs=[pl.BlockSpec((1, L, F), lambda b, j: (b, 0, 0)),
                  pl.BlockSpec((1, BM, F), lambda b, j: (b, j, 0)),
                  pl.BlockSpec((1, 1, BM), lambda b, j: (b, 0, j))],
        out_specs=pl.BlockSpec((1, L, BM), lambda b, j: (b, 0, j)),
    )(hp, triple_repr, labels3)


def _k3_body(tp_hbm, head_hbm, tail_hbm, tlab_hbm, clab_hbm, dist_hbm,
             out_hbm,
             head_v, tail_v, tlab_v, tp_v, clab_v, dist_v, upd_v, ns_v,
             priv_v, acc_v):
    c = lax.axis_index("c")
    s = lax.axis_index("s")

    @pl.when(s < 8)
    def _():
        pair = c * 8 + s
        b = pair // 4
        l = pair % 4
        pltpu.sync_copy(head_hbm.at[b], head_v)
        pltpu.sync_copy(tail_hbm.at[b], tail_v)
        pltpu.sync_copy(tlab_hbm.at[b], tlab_v)
        pltpu.sync_copy(tp_hbm.at[b, l], tp_v)
        pltpu.sync_copy(clab_hbm.at[b], clab_v)
        pltpu.sync_copy(dist_hbm.at[b], dist_v)
        one = jnp.ones((16,), jnp.float32)
        zero = jnp.zeros((16,), jnp.float32)
        lane = lax.iota(jnp.int32, 16)
        for i in range(Mc // 16):
            sl = pl.ds(i * 16, 16)
            f0 = jnp.where(dist_v[sl] == 0, one, zero)
            acc_v[sl] = f0
            ns_v[sl] = jnp.where(clab_v[sl] == -1, zero, f0)
        for _hop in range(2):
            for i in range(Mt // 16):
                sl = pl.ds(i * 16, 16)
                hs = plsc.load_gather(ns_v, [head_v[sl]])
                upd = hs * GAMMA + tp_v[sl]
                upd_v[sl] = jnp.where(tlab_v[sl] == -1, zero, upd)

            def zbody(i, carry):
                priv_v[pl.ds(i * 16, 16)] = zero
                return carry

            lax.fori_loop(0, 16 * Mc // 16, zbody, 0)

            def scbody(i, carry):
                sl = pl.ds(i * 16, 16)
                addr = lane * Mc + tail_v[sl]
                cur = plsc.load_gather(priv_v, [addr])
                plsc.store_scatter(priv_v, [addr],
                                   jnp.maximum(cur, upd_v[sl]))
                return carry

            lax.fori_loop(0, Mt // 16, scbody, 0)

            def cbody(i, carry):
                sl16 = pl.ds(i * 16, 16)
                m = priv_v[sl16]
                for j in range(1, 16):
                    m = jnp.maximum(m, priv_v[pl.ds(j * Mc + i * 16, 16)])
                o = jnp.where(clab_v[sl16] == -1, zero, m)
                ns_v[sl16] = o
                acc_v[sl16] = acc_v[sl16] + o
                return carry

            lax.fori_loop(0, Mc // 16, cbody, 0)
        mxv = acc_v[pl.ds(0, 16)]
        for i in range(1, Mc // 16):
            mxv = jnp.maximum(mxv, acc_v[pl.ds(i * 16, 16)])
        m = jnp.max(mxv)
        sv = jnp.zeros((16,), jnp.float32)
        for i in range(Mc // 16):
            sl = pl.ds(i * 16, 16)
            e = jnp.exp(acc_v[sl] - m)
            acc_v[sl] = e
            sv = sv + e
        inv_v = one / (zero + jnp.sum(sv))
        for i in range(Mc // 16):
            sl = pl.ds(i * 16, 16)
            acc_v[sl] = acc_v[sl] * inv_v
        pltpu.sync_copy(acc_v, out_hbm.at[b, l])


def _k3(tp, head_idx, tail_idx, triple_labels, concept_labels, distances):
    mesh = plsc.VectorSubcoreMesh(core_axis_name="c", subcore_axis_name="s")
    f = functools.partial(
        pl.kernel, _k3_body, mesh=mesh,
        compiler_params=pltpu.CompilerParams(needs_layout_passes=False),
        out_type=jax.ShapeDtypeStruct((B, L, Mc), jnp.float32),
        scratch_types=[
            pltpu.VMEM((Mt,), jnp.int32),
            pltpu.VMEM((Mt,), jnp.int32),
            pltpu.VMEM((Mt,), jnp.int32),
            pltpu.VMEM((Mt,), jnp.float32),
            pltpu.VMEM((Mc,), jnp.int32),
            pltpu.VMEM((Mc,), jnp.int32),
            pltpu.VMEM((Mt,), jnp.float32),
            pltpu.VMEM((Mc,), jnp.float32),
            pltpu.VMEM((16 * Mc,), jnp.float32),
            pltpu.VMEM((Mc,), jnp.float32),
        ],
    )()
    return f(tp, head_idx, tail_idx, triple_labels, concept_labels, distances)


def _k4_body(cp_hbm, gate_hbm, vm_hbm, mm_hbm, lm_hbm,
             probs_hbm, cpv_hbm, isc_hbm,
             table_v, gate_v, vm_v, mm_v, lm_v, pr_v, cv_v,
             stat_f, stat_i, part_f, part_i, shared_f, shared_i):
    c = lax.axis_index("c")
    s = lax.axis_index("s")
    pair = c * 8 + s // 2
    half = s % 2
    b = pair // 4
    l = pair % 4
    base = pl.multiple_of(half * HALF, 32)
    pltpu.sync_copy(cp_hbm.at[b, l], table_v)
    pltpu.sync_copy(gate_hbm, gate_v)
    lane = lax.iota(jnp.int32, 16)
    gv = gate_v[...]
    g = jnp.sum(jnp.where(lane == pair, gv, 0.0))
    neg1 = jnp.zeros((16,), jnp.float32) - 1.0
    zeroi = jnp.zeros((16,), jnp.int32)
    zero = jnp.zeros((16,), jnp.float32)

    def blend(off, nvec, kcar):
        def inner(k, kc):
            rmxP, riP, rmxL, riL = kc
            sl = pl.ds(k * 16, 16)
            cpx = plsc.load_gather(table_v, [vm_v[sl]])
            cpx = jnp.where(mm_v[sl] == 0, zero, cpx)
            lmv = lm_v[sl]
            p = g * cpx + (1.0 - g) * lmv
            cv_v[sl] = cpx
            pr_v[sl] = p
            vidx = (off + k * 16) + lane
            bP = p > rmxP
            rmxP = jnp.where(bP, p, rmxP)
            riP = jnp.where(bP, vidx, riP)
            bL = lmv > rmxL
            rmxL = jnp.where(bL, lmv, rmxL)
            riL = jnp.where(bL, vidx, riL)
            return (rmxP, riP, rmxL, riL)

        return lax.fori_loop(0, nvec, inner, kcar)

    def big(i, car):
        off = pl.multiple_of(base + i * CB, 8)
        pltpu.sync_copy(vm_hbm.at[pl.ds(b * V + off, CB)], vm_v)
        pltpu.sync_copy(mm_hbm.at[pl.ds(b * V + off, CB)], mm_v)
        pltpu.sync_copy(lm_hbm.at[pl.ds(pair * V + off, CB)], lm_v)
        car2 = blend(off, CB // 16, car)
        pltpu.sync_copy(pr_v, probs_hbm.at[pl.ds(pair * V + off, CB)])
        pltpu.sync_copy(cv_v, cpv_hbm.at[pl.ds(pair * V + off, CB)])
        return car2

    car = lax.fori_loop(0, NFULL, big, (neg1, zeroi, neg1, zeroi))

    toff = pl.multiple_of(base + NFULL * CB, 8)
    pltpu.sync_copy(vm_hbm.at[pl.ds(b * V + toff, TAIL)],
                    vm_v.at[pl.ds(0, TAIL)])
    pltpu.sync_copy(mm_hbm.at[pl.ds(b * V + toff, TAIL)],
                    mm_v.at[pl.ds(0, TAIL)])
    pltpu.sync_copy(lm_hbm.at[pl.ds(pair * V + toff, TAIL)],
                    lm_v.at[pl.ds(0, TAIL)])
    rmxP, riP, rmxL, riL = blend(toff, TAIL // 16, car)

    @pl.when(half == 0)
    def _():
        pltpu.sync_copy(pr_v.at[pl.ds(0, TW0)],
                        probs_hbm.at[pl.ds(pair * V + toff, TW0)])
        pltpu.sync_copy(cv_v.at[pl.ds(0, TW0)],
                        cpv_hbm.at[pl.ds(pair * V + toff, TW0)])

    @pl.when(half == 1)
    def _():
        pltpu.sync_copy(pr_v.at[pl.ds(0, TAIL)],
                        probs_hbm.at[pl.ds(pair * V + toff, TAIL)])
        pltpu.sync_copy(cv_v.at[pl.ds(0, TAIL)],
                        cpv_hbm.at[pl.ds(pair * V + toff, TAIL)])

    mP = jnp.max(rmxP)
    iP = jnp.min(jnp.where(rmxP == mP, riP, BIGI))
    mL = jnp.max(rmxL)
    iL = jnp.min(jnp.where(rmxL == mL, riL, BIGI))
    stat_f[...] = jnp.where(lane == 0, mP, jnp.where(lane == 1, mL, 0.0))
    stat_i[...] = jnp.where(lane == 0, iP, jnp.where(lane == 1, iL, 0))
    pltpu.sync_copy(stat_f, shared_f.at[s])
    pltpu.sync_copy(stat_i, shared_i.at[s])
    plsc.subcore_barrier()

    @pl.when(half == 0)
    def _():
        pltpu.sync_copy(shared_f.at[s + 1], part_f)
        pltpu.sync_copy(shared_i.at[s + 1], part_i)
        pf = part_f[...]
        pi = part_i[...]
        giP = jnp.where(pf[0] > mP, pi[0], iP)
        giL = jnp.where(pf[1] > mL, pi[1], iL)
        isc = jnp.where(giP != giL, jnp.int32(1), jnp.int32(0))
        stat_i[...] = zeroi + isc
        pltpu.sync_copy(stat_i, isc_hbm.at[pair])


def _k4(cp, gate16f, vm_p, mm_p, lm_p):
    mesh = plsc.VectorSubcoreMesh(core_axis_name="c", subcore_axis_name="s")
    f = functools.partial(
        pl.kernel, _k4_body, mesh=mesh,
        compiler_params=pltpu.CompilerParams(needs_layout_passes=False),
        out_type=[jax.ShapeDtypeStruct((B * L * V,), jnp.float32),
                  jax.ShapeDtypeStruct((B * L * V,), jnp.float32),
                  jax.ShapeDtypeStruct((16, 16), jnp.int32)],
        scratch_types=[
            pltpu.VMEM((Mc,), jnp.float32),
            pltpu.VMEM((16,), jnp.float32),
            pltpu.VMEM((CB,), jnp.int32),
            pltpu.VMEM((CB,), jnp.int32),
            pltpu.VMEM((CB,), jnp.float32),
            pltpu.VMEM((CB,), jnp.float32),
            pltpu.VMEM((CB,), jnp.float32),
            pltpu.VMEM((16,), jnp.float32),
            pltpu.VMEM((16,), jnp.int32),
            pltpu.VMEM((16,), jnp.float32),
            pltpu.VMEM((16,), jnp.int32),
            pltpu.VMEM_SHARED((16, 16), jnp.float32),
            pltpu.VMEM_SHARED((16, 16), jnp.int32),
        ],
    )()
    return f(cp, gate16f, vm_p, mm_p, lm_p)


def kernel(lm_hidden_states, lm_probs, triple_repr, triple_labels,
           concept_labels, distances, head_idx, tail_idx, vocab_map,
           map_mask, W_triple, W_gate, b_gate):
    h16 = lm_hidden_states.reshape(16, E)
    bg = b_gate.reshape(1, 1)
    h_proj, gate16 = _k1(h16, W_triple, W_gate.T, bg)
    hp = h_proj.reshape(B, L, F)
    labels3 = triple_labels.reshape(B, 1, Mt)
    triple_prob = _k2(hp, triple_repr, labels3)
    cp = _k3(triple_prob, head_idx, tail_idx, triple_labels,
             concept_labels, distances)
    probs_f, cpv_f, isc16 = _k4(cp, gate16.reshape(16),
                                vocab_map.reshape(B * V),
                                map_mask.reshape(B * V),
                                lm_probs.reshape(B * L * V))
    probs = probs_f.reshape(B, L, V)
    cpv = cpv_f.reshape(B, L, V)
    gate = gate16.reshape(B, L, 1)
    is_concept = isc16[:, 0].reshape(B, L)
    return probs, gate, cpv, triple_prob, is_concept


# speedup vs baseline: 1.2866x; 1.2866x over previous
import functools

import jax
import jax.numpy as jnp
from jax import lax
from jax.experimental import pallas as pl
from jax.experimental.pallas import tpu as pltpu
from jax.experimental.pallas import tpu_sc as plsc

B, L, E, Mt, Mc, V = 4, 4, 1024, 1024, 512, 50000
F = 3 * E
GAMMA = 0.8
VP = 50176
HALF = VP // 2
NV = HALF // 16
BM = 512
BIGI = 2 ** 30


def _k1_body(h_ref, wt_ref, wg_ref, bg_ref, hp_ref, gate_ref):
    h = h_ref[...]
    hp_ref[...] = jnp.dot(h, wt_ref[...], preferred_element_type=jnp.float32)
    z = jnp.dot(h, wg_ref[...], preferred_element_type=jnp.float32)
    gate_ref[...] = jax.nn.sigmoid(z + bg_ref[0, 0])


def _k1(h16, W_triple, Wg_t, bg):
    return pl.pallas_call(
        _k1_body,
        out_shape=[jax.ShapeDtypeStruct((16, F), jnp.float32),
                   jax.ShapeDtypeStruct((16, 1), jnp.float32)],
        in_specs=[pl.BlockSpec((16, E), lambda: (0, 0)),
                  pl.BlockSpec((E, F), lambda: (0, 0)),
                  pl.BlockSpec((E, 1), lambda: (0, 0)),
                  pl.BlockSpec(memory_space=pltpu.SMEM)],
        out_specs=[pl.BlockSpec((16, F), lambda: (0, 0)),
                   pl.BlockSpec((16, 1), lambda: (0, 0))],
    )(h16, W_triple, Wg_t, bg)


def _k2_body(hp_ref, tr_ref, lab_ref, out_ref):
    hp = hp_ref[0]
    tr = tr_ref[0]
    logits = lax.dot_general(hp, tr, (((1,), (1,)), ((), ())),
                             preferred_element_type=jnp.float32)
    prob = jax.nn.sigmoid(logits)
    lab = lab_ref[0]
    out_ref[0] = jnp.where(lab == -1, 0.0, prob)


def _k2(hp, triple_repr, labels3):
    return pl.pallas_call(
        _k2_body,
        grid=(B, Mt // BM),
        out_shape=jax.ShapeDtypeStruct((B, L, Mt), jnp.float32),
        in_specs=[pl.BlockSpec((1, L, F), lambda b, j: (b, 0, 0)),
                  pl.BlockSpec((1, BM, F), lambda b, j: (b, j, 0)),
                  pl.BlockSpec((1, 1, BM), lambda b, j: (b, 0, j))],
        out_specs=pl.BlockSpec((1, L, BM), lambda b, j: (b, 0, j)),
    )(hp, triple_repr, labels3)


def _k3_body(tp_hbm, head_hbm, tail_hbm, tlab_hbm, clab_hbm, dist_hbm,
             out_hbm,
             head_v, tail_v, tlab_v, tp_v, clab_v, dist_v, upd_v, ns_v,
             priv_v, acc_v, sem):
    c = lax.axis_index("c")
    s = lax.axis_index("s")

    @pl.when(s < 8)
    def _():
        pair = c * 8 + s
        b = pair // 4
        l = pair % 4
        d1 = pltpu.async_copy(head_hbm.at[b], head_v, sem)
        d2 = pltpu.async_copy(tail_hbm.at[b], tail_v, sem)
        d3 = pltpu.async_copy(tlab_hbm.at[b], tlab_v, sem)
        d4 = pltpu.async_copy(tp_hbm.at[b, l], tp_v, sem)
        d5 = pltpu.async_copy(clab_hbm.at[b], clab_v, sem)
        d6 = pltpu.async_copy(dist_hbm.at[b], dist_v, sem)
        d1.wait()
        d2.wait()
        d3.wait()
        d4.wait()
        d5.wait()
        d6.wait()
        one = jnp.ones((16,), jnp.float32)
        zero = jnp.zeros((16,), jnp.float32)
        lane = lax.iota(jnp.int32, 16)
        for i in range(Mc // 16):
            sl = pl.ds(i * 16, 16)
            f0 = jnp.where(dist_v[sl] == 0, one, zero)
            acc_v[sl] = f0
            ns_v[sl] = jnp.where(clab_v[sl] == -1, zero, f0)
        for _hop in range(2):
            for i in range(Mt // 16):
                sl = pl.ds(i * 16, 16)
                hs = plsc.load_gather(ns_v, [head_v[sl]])
                upd = hs * GAMMA + tp_v[sl]
                upd_v[sl] = jnp.where(tlab_v[sl] == -1, zero, upd)

            def zbody(i, carry):
                priv_v[pl.ds(i * 16, 16)] = zero
                return carry

            lax.fori_loop(0, 16 * Mc // 16, zbody, 0)

            def scbody(i, carry):
                sl = pl.ds(i * 16, 16)
                addr = lane * Mc + tail_v[sl]
                cur = plsc.load_gather(priv_v, [addr])
                plsc.store_scatter(priv_v, [addr],
                                   jnp.maximum(cur, upd_v[sl]))
                return carry

            lax.fori_loop(0, Mt // 16, scbody, 0)

            def cbody(i, carry):
                sl16 = pl.ds(i * 16, 16)
                m = priv_v[sl16]
                for j in range(1, 16):
                    m = jnp.maximum(m, priv_v[pl.ds(j * Mc + i * 16, 16)])
                o = jnp.where(clab_v[sl16] == -1, zero, m)
                ns_v[sl16] = o
                acc_v[sl16] = acc_v[sl16] + o
                return carry

            lax.fori_loop(0, Mc // 16, cbody, 0)
        mxv = acc_v[pl.ds(0, 16)]
        for i in range(1, Mc // 16):
            mxv = jnp.maximum(mxv, acc_v[pl.ds(i * 16, 16)])
        m = jnp.max(mxv)
        sv = jnp.zeros((16,), jnp.float32)
        for i in range(Mc // 16):
            sl = pl.ds(i * 16, 16)
            e = jnp.exp(acc_v[sl] - m)
            acc_v[sl] = e
            sv = sv + e
        inv_v = one / (zero + jnp.sum(sv))
        for i in range(Mc // 16):
            sl = pl.ds(i * 16, 16)
            acc_v[sl] = acc_v[sl] * inv_v
        pltpu.sync_copy(acc_v, out_hbm.at[b, l])


def _k3(tp, head_idx, tail_idx, triple_labels, concept_labels, distances):
    mesh = plsc.VectorSubcoreMesh(core_axis_name="c", subcore_axis_name="s")
    f = functools.partial(
        pl.kernel, _k3_body, mesh=mesh,
        compiler_params=pltpu.CompilerParams(needs_layout_passes=False),
        out_type=jax.ShapeDtypeStruct((B, L, Mc), jnp.float32),
        scratch_types=[
            pltpu.VMEM((Mt,), jnp.int32),
            pltpu.VMEM((Mt,), jnp.int32),
            pltpu.VMEM((Mt,), jnp.int32),
            pltpu.VMEM((Mt,), jnp.float32),
            pltpu.VMEM((Mc,), jnp.int32),
            pltpu.VMEM((Mc,), jnp.int32),
            pltpu.VMEM((Mt,), jnp.float32),
            pltpu.VMEM((Mc,), jnp.float32),
            pltpu.VMEM((16 * Mc,), jnp.float32),
            pltpu.VMEM((Mc,), jnp.float32),
            pltpu.SemaphoreType.DMA,
        ],
    )()
    return f(tp, head_idx, tail_idx, triple_labels, concept_labels, distances)


def _k4_body(cp_hbm, gate_hbm, vm_hbm, mm_hbm, lm_hbm,
             probs_hbm, cpv_hbm, isc_hbm,
             table_v, gate_v, vm_v, mm_v, lm_v, pr_v, cv_v,
             stat_f, stat_i, part_f, part_i, shared_f, shared_i, sem):
    c = lax.axis_index("c")
    s = lax.axis_index("s")
    pair = c * 8 + s // 2
    half = s % 2
    b = pair // 4
    l = pair % 4
    base = pl.multiple_of(half * HALF, 32)
    d1 = pltpu.async_copy(vm_hbm.at[b, pl.ds(base, HALF)], vm_v, sem)
    d2 = pltpu.async_copy(mm_hbm.at[b, pl.ds(base, HALF)], mm_v, sem)
    d3 = pltpu.async_copy(lm_hbm.at[b, l, pl.ds(base, HALF)], lm_v, sem)
    pltpu.sync_copy(cp_hbm.at[b, l], table_v)
    pltpu.sync_copy(gate_hbm, gate_v)
    lane = lax.iota(jnp.int32, 16)
    gv = gate_v[...]
    g = jnp.sum(jnp.where(lane == pair, gv, 0.0))
    neg1 = jnp.zeros((16,), jnp.float32) - 1.0
    zeroi = jnp.zeros((16,), jnp.int32)
    zero = jnp.zeros((16,), jnp.float32)
    d1.wait()
    d2.wait()
    d3.wait()

    def inner(k, kc):
        rmxP, riP, rmxL, riL = kc
        sl = pl.ds(k * 16, 16)
        cpx = plsc.load_gather(table_v, [vm_v[sl]])
        cpx = jnp.where(mm_v[sl] == 0, zero, cpx)
        lmv = lm_v[sl]
        p = g * cpx + (1.0 - g) * lmv
        cv_v[sl] = cpx
        pr_v[sl] = p
        vidx = (base + k * 16) + lane
        bP = p > rmxP
        rmxP = jnp.where(bP, p, rmxP)
        riP = jnp.where(bP, vidx, riP)
        bL = lmv > rmxL
        rmxL = jnp.where(bL, lmv, rmxL)
        riL = jnp.where(bL, vidx, riL)
        return (rmxP, riP, rmxL, riL)

    rmxP, riP, rmxL, riL = lax.fori_loop(
        0, NV, inner, (neg1, zeroi, neg1, zeroi))

    o1 = pltpu.async_copy(pr_v, probs_hbm.at[b, l, pl.ds(base, HALF)], sem)
    o2 = pltpu.async_copy(cv_v, cpv_hbm.at[b, l, pl.ds(base, HALF)], sem)

    mP = jnp.max(rmxP)
    iP = jnp.min(jnp.where(rmxP == mP, riP, BIGI))
    mL = jnp.max(rmxL)
    iL = jnp.min(jnp.where(rmxL == mL, riL, BIGI))
    stat_f[...] = jnp.where(lane == 0, mP, jnp.where(lane == 1, mL, 0.0))
    stat_i[...] = jnp.where(lane == 0, iP, jnp.where(lane == 1, iL, 0))
    pltpu.sync_copy(stat_f, shared_f.at[s])
    pltpu.sync_copy(stat_i, shared_i.at[s])
    plsc.subcore_barrier()

    @pl.when(half == 0)
    def _():
        pltpu.sync_copy(shared_f.at[s + 1], part_f)
        pltpu.sync_copy(shared_i.at[s + 1], part_i)
        pf = part_f[...]
        pi = part_i[...]
        giP = jnp.where(pf[0] > mP, pi[0], iP)
        giL = jnp.where(pf[1] > mL, pi[1], iL)
        isc = jnp.where(giP != giL, jnp.int32(1), jnp.int32(0))
        stat_i[...] = zeroi + isc
        pltpu.sync_copy(stat_i, isc_hbm.at[pair])

    o1.wait()
    o2.wait()


def _k4(cp, gate16f, vm_p, mm_p, lm_p):
    mesh = plsc.VectorSubcoreMesh(core_axis_name="c", subcore_axis_name="s")
    f = functools.partial(
        pl.kernel, _k4_body, mesh=mesh,
        compiler_params=pltpu.CompilerParams(needs_layout_passes=False),
        out_type=[jax.ShapeDtypeStruct((B, L, VP), jnp.float32),
                  jax.ShapeDtypeStruct((B, L, VP), jnp.float32),
                  jax.ShapeDtypeStruct((16, 16), jnp.int32)],
        scratch_types=[
            pltpu.VMEM((Mc,), jnp.float32),
            pltpu.VMEM((16,), jnp.float32),
            pltpu.VMEM((HALF,), jnp.int32),
            pltpu.VMEM((HALF,), jnp.int32),
            pltpu.VMEM((HALF,), jnp.float32),
            pltpu.VMEM((HALF,), jnp.float32),
            pltpu.VMEM((HALF,), jnp.float32),
            pltpu.VMEM((16,), jnp.float32),
            pltpu.VMEM((16,), jnp.int32),
            pltpu.VMEM((16,), jnp.float32),
            pltpu.VMEM((16,), jnp.int32),
            pltpu.VMEM_SHARED((16, 16), jnp.float32),
            pltpu.VMEM_SHARED((16, 16), jnp.int32),
            pltpu.SemaphoreType.DMA,
        ],
    )()
    return f(cp, gate16f, vm_p, mm_p, lm_p)


def kernel(lm_hidden_states, lm_probs, triple_repr, triple_labels,
           concept_labels, distances, head_idx, tail_idx, vocab_map,
           map_mask, W_triple, W_gate, b_gate):
    h16 = lm_hidden_states.reshape(16, E)
    bg = b_gate.reshape(1, 1)
    h_proj, gate16 = _k1(h16, W_triple, W_gate.T, bg)
    hp = h_proj.reshape(B, L, F)
    labels3 = triple_labels.reshape(B, 1, Mt)
    triple_prob = _k2(hp, triple_repr, labels3)
    cp = _k3(triple_prob, head_idx, tail_idx, triple_labels,
             concept_labels, distances)
    vm_p = jnp.pad(vocab_map, ((0, 0), (0, VP - V)))
    mm_p = jnp.pad(map_mask, ((0, 0), (0, VP - V)))
    lm_p = jnp.pad(lm_probs, ((0, 0), (0, 0), (0, VP - V)))
    probs_p, cpv_p, isc16 = _k4(cp, gate16.reshape(16), vm_p, mm_p, lm_p)
    probs = probs_p[:, :, :V]
    cpv = cpv_p[:, :, :V]
    gate = gate16.reshape(B, L, 1)
    is_concept = isc16[:, 0].reshape(B, L)
    return probs, gate, cpv, triple_prob, is_concept


# speedup vs baseline: 1.3224x; 1.0279x over previous
import functools

import jax
import jax.numpy as jnp
from jax import lax
from jax.experimental import pallas as pl
from jax.experimental.pallas import tpu as pltpu
from jax.experimental.pallas import tpu_sc as plsc

B, L, E, Mt, Mc, V = 4, 4, 1024, 1024, 512, 50000
F = 3 * E
GAMMA = 0.8
VP = 50176
HALF = VP // 2
NV = HALF // 16
BM = 512
BIGI = 2 ** 30


def _k1_body(h_ref, wt_ref, wg_ref, bg_ref, hp_ref, gate_ref):
    h = h_ref[...]
    hp_ref[...] = jnp.dot(h, wt_ref[...], preferred_element_type=jnp.float32)
    z = jnp.dot(h, wg_ref[...], preferred_element_type=jnp.float32)
    gate_ref[...] = jax.nn.sigmoid(z + bg_ref[0, 0])


def _k1(h16, W_triple, Wg_t, bg):
    return pl.pallas_call(
        _k1_body,
        out_shape=[jax.ShapeDtypeStruct((16, F), jnp.float32),
                   jax.ShapeDtypeStruct((16, 1), jnp.float32)],
        in_specs=[pl.BlockSpec((16, E), lambda: (0, 0)),
                  pl.BlockSpec((E, F), lambda: (0, 0)),
                  pl.BlockSpec((E, 1), lambda: (0, 0)),
                  pl.BlockSpec(memory_space=pltpu.SMEM)],
        out_specs=[pl.BlockSpec((16, F), lambda: (0, 0)),
                   pl.BlockSpec((16, 1), lambda: (0, 0))],
    )(h16, W_triple, Wg_t, bg)


def _k2_body(hp_ref, tr_ref, lab_ref, out_ref):
    hp = hp_ref[0]
    tr = tr_ref[0]
    logits = lax.dot_general(hp, tr, (((1,), (1,)), ((), ())),
                             preferred_element_type=jnp.float32)
    prob = jax.nn.sigmoid(logits)
    lab = lab_ref[0]
    out_ref[0] = jnp.where(lab == -1, 0.0, prob)


def _k2(hp, triple_repr, labels3):
    return pl.pallas_call(
        _k2_body,
        grid=(B, Mt // BM),
        out_shape=jax.ShapeDtypeStruct((B, L, Mt), jnp.float32),
        in_specs=[pl.BlockSpec((1, L, F), lambda b, j: (b, 0, 0)),
                  pl.BlockSpec((1, BM, F), lambda b, j: (b, j, 0)),
                  pl.BlockSpec((1, 1, BM), lambda b, j: (b, 0, j))],
        out_specs=pl.BlockSpec((1, L, BM), lambda b, j: (b, 0, j)),
    )(hp, triple_repr, labels3)


def _k3_body(tp_hbm, head_hbm, tail_hbm, tlab_hbm, clab_hbm, dist_hbm,
             out_hbm,
             head_v, tail_v, tlab_v, tp_v, clab_v, dist_v, upd_v, ns_v,
             priv_v, acc_v, sem):
    c = lax.axis_index("c")
    s = lax.axis_index("s")

    @pl.when(s < 8)
    def _():
        pair = c * 8 + s
        b = pair // 4
        l = pair % 4
        d1 = pltpu.async_copy(head_hbm.at[b], head_v, sem)
        d2 = pltpu.async_copy(tail_hbm.at[b], tail_v, sem)
        d3 = pltpu.async_copy(tlab_hbm.at[b], tlab_v, sem)
        d4 = pltpu.async_copy(tp_hbm.at[b, l], tp_v, sem)
        d5 = pltpu.async_copy(clab_hbm.at[b], clab_v, sem)
        d6 = pltpu.async_copy(dist_hbm.at[b], dist_v, sem)
        d1.wait()
        d2.wait()
        d3.wait()
        d4.wait()
        d5.wait()
        d6.wait()
        one = jnp.ones((16,), jnp.float32)
        zero = jnp.zeros((16,), jnp.float32)
        lane = lax.iota(jnp.int32, 16)
        for i in range(Mc // 16):
            sl = pl.ds(i * 16, 16)
            f0 = jnp.where(dist_v[sl] == 0, one, zero)
            acc_v[sl] = f0
            ns_v[sl] = jnp.where(clab_v[sl] == -1, zero, f0)
        for _hop in range(2):
            for i in range(Mt // 16):
                sl = pl.ds(i * 16, 16)
                hs = plsc.load_gather(ns_v, [head_v[sl]])
                upd = hs * GAMMA + tp_v[sl]
                upd_v[sl] = jnp.where(tlab_v[sl] == -1, zero, upd)

            def zbody(i, carry):
                priv_v[pl.ds(i * 16, 16)] = zero
                return carry

            lax.fori_loop(0, 16 * Mc // 16, zbody, 0)

            def scbody(i, carry):
                sl = pl.ds(i * 16, 16)
                addr = lane * Mc + tail_v[sl]
                cur = plsc.load_gather(priv_v, [addr])
                plsc.store_scatter(priv_v, [addr],
                                   jnp.maximum(cur, upd_v[sl]))
                return carry

            lax.fori_loop(0, Mt // 16, scbody, 0)

            def cbody(i, carry):
                sl16 = pl.ds(i * 16, 16)
                m = priv_v[sl16]
                for j in range(1, 16):
                    m = jnp.maximum(m, priv_v[pl.ds(j * Mc + i * 16, 16)])
                o = jnp.where(clab_v[sl16] == -1, zero, m)
                ns_v[sl16] = o
                acc_v[sl16] = acc_v[sl16] + o
                return carry

            lax.fori_loop(0, Mc // 16, cbody, 0)
        mxv = acc_v[pl.ds(0, 16)]
        for i in range(1, Mc // 16):
            mxv = jnp.maximum(mxv, acc_v[pl.ds(i * 16, 16)])
        m = jnp.max(mxv)
        sv = jnp.zeros((16,), jnp.float32)
        for i in range(Mc // 16):
            sl = pl.ds(i * 16, 16)
            e = jnp.exp(acc_v[sl] - m)
            acc_v[sl] = e
            sv = sv + e
        inv_v = one / (zero + jnp.sum(sv))
        for i in range(Mc // 16):
            sl = pl.ds(i * 16, 16)
            acc_v[sl] = acc_v[sl] * inv_v
        pltpu.sync_copy(acc_v, out_hbm.at[b, l])


def _k3(tp, head_idx, tail_idx, triple_labels, concept_labels, distances):
    mesh = plsc.VectorSubcoreMesh(core_axis_name="c", subcore_axis_name="s")
    f = functools.partial(
        pl.kernel, _k3_body, mesh=mesh,
        compiler_params=pltpu.CompilerParams(needs_layout_passes=False),
        out_type=jax.ShapeDtypeStruct((B, L, Mc), jnp.float32),
        scratch_types=[
            pltpu.VMEM((Mt,), jnp.int32),
            pltpu.VMEM((Mt,), jnp.int32),
            pltpu.VMEM((Mt,), jnp.int32),
            pltpu.VMEM((Mt,), jnp.float32),
            pltpu.VMEM((Mc,), jnp.int32),
            pltpu.VMEM((Mc,), jnp.int32),
            pltpu.VMEM((Mt,), jnp.float32),
            pltpu.VMEM((Mc,), jnp.float32),
            pltpu.VMEM((16 * Mc,), jnp.float32),
            pltpu.VMEM((Mc,), jnp.float32),
            pltpu.SemaphoreType.DMA,
        ],
    )()
    return f(tp, head_idx, tail_idx, triple_labels, concept_labels, distances)


def _k4_body(cp_hbm, vm_hbm, mm_hbm,
             cpv_hbm,
             table_v, vm_v, mm_v, cv_v, sem):
    c = lax.axis_index("c")
    s = lax.axis_index("s")
    pair = c * 8 + s // 2
    half = s % 2
    b = pair // 4
    l = pair % 4
    base = pl.multiple_of(half * HALF, 32)
    d1 = pltpu.async_copy(vm_hbm.at[b, pl.ds(base, HALF)], vm_v, sem)
    d2 = pltpu.async_copy(mm_hbm.at[b, pl.ds(base, HALF)], mm_v, sem)
    pltpu.sync_copy(cp_hbm.at[b, l], table_v)
    zero = jnp.zeros((16,), jnp.float32)
    d1.wait()
    d2.wait()

    def inner(k, carry):
        sl = pl.ds(k * 16, 16)
        cpx = plsc.load_gather(table_v, [vm_v[sl]])
        cv_v[sl] = jnp.where(mm_v[sl] == 0, zero, cpx)
        return carry

    lax.fori_loop(0, NV, inner, 0)
    pltpu.sync_copy(cv_v, cpv_hbm.at[b, l, pl.ds(base, HALF)])


def _k4(cp, vm_p, mm_p):
    mesh = plsc.VectorSubcoreMesh(core_axis_name="c", subcore_axis_name="s")
    f = functools.partial(
        pl.kernel, _k4_body, mesh=mesh,
        compiler_params=pltpu.CompilerParams(needs_layout_passes=False),
        out_type=jax.ShapeDtypeStruct((B, L, VP), jnp.float32),
        scratch_types=[
            pltpu.VMEM((Mc,), jnp.float32),
            pltpu.VMEM((HALF,), jnp.int32),
            pltpu.VMEM((HALF,), jnp.int32),
            pltpu.VMEM((HALF,), jnp.float32),
            pltpu.SemaphoreType.DMA,
        ],
    )()
    return f(cp, vm_p, mm_p)


def _k5_body(cpvp_ref, lm_ref, gate_ref, probs_ref, cpv_ref, isc_ref):
    cpx = cpvp_ref[0, :, :V]
    lm = lm_ref[0]
    gcol = gate_ref[0, 0].reshape(L, 1)
    p = gcol * cpx + (1.0 - gcol) * lm
    probs_ref[0] = p
    cpv_ref[0] = cpx
    iota = lax.broadcasted_iota(jnp.int32, (L, V), 1)
    mP = jnp.max(p, axis=1, keepdims=True)
    mL = jnp.max(lm, axis=1, keepdims=True)
    iP = jnp.min(jnp.where(p == mP, iota, BIGI), axis=1)
    iL = jnp.min(jnp.where(lm == mL, iota, BIGI), axis=1)
    isc_ref[0, 0] = jnp.where(iP != iL, 1, 0).astype(jnp.int32)


def _k5(cpv_p, lm_probs, gate_bl):
    return pl.pallas_call(
        _k5_body,
        grid=(B,),
        out_shape=[jax.ShapeDtypeStruct((B, L, V), jnp.float32),
                   jax.ShapeDtypeStruct((B, L, V), jnp.float32),
                   jax.ShapeDtypeStruct((B, 1, L), jnp.int32)],
        in_specs=[pl.BlockSpec((1, L, VP), lambda b: (b, 0, 0)),
                  pl.BlockSpec((1, L, V), lambda b: (b, 0, 0)),
                  pl.BlockSpec((1, 1, L), lambda b: (b, 0, 0))],
        out_specs=[pl.BlockSpec((1, L, V), lambda b: (b, 0, 0)),
                   pl.BlockSpec((1, L, V), lambda b: (b, 0, 0)),
                   pl.BlockSpec((1, 1, L), lambda b: (b, 0, 0))],
    )(cpv_p, lm_probs, gate_bl)


def kernel(lm_hidden_states, lm_probs, triple_repr, triple_labels,
           concept_labels, distances, head_idx, tail_idx, vocab_map,
           map_mask, W_triple, W_gate, b_gate):
    h16 = lm_hidden_states.reshape(16, E)
    bg = b_gate.reshape(1, 1)
    h_proj, gate16 = _k1(h16, W_triple, W_gate.T, bg)
    hp = h_proj.reshape(B, L, F)
    labels3 = triple_labels.reshape(B, 1, Mt)
    triple_prob = _k2(hp, triple_repr, labels3)
    cp = _k3(triple_prob, head_idx, tail_idx, triple_labels,
             concept_labels, distances)
    vm_p = jnp.pad(vocab_map, ((0, 0), (0, VP - V)))
    mm_p = jnp.pad(map_mask, ((0, 0), (0, VP - V)))
    cpv_p = _k4(cp, vm_p, mm_p)
    probs, cpv, isc = _k5(cpv_p, lm_probs, gate16.reshape(B, 1, L))
    gate = gate16.reshape(B, L, 1)
    return probs, gate, cpv, triple_prob, isc.reshape(B, L)


# speedup vs baseline: 1.3307x; 1.0063x over previous
import functools

import jax
import jax.numpy as jnp
from jax import lax
from jax.experimental import pallas as pl
from jax.experimental.pallas import tpu as pltpu
from jax.experimental.pallas import tpu_sc as plsc

B, L, E, Mt, Mc, V = 4, 4, 1024, 1024, 512, 50000
F = 3 * E
GAMMA = 0.8
VP = 50176
HALF = VP // 2
NV = HALF // 16
CH = HALF // 2
NCH = CH // 16
BM = 512
BIGI = 2 ** 30


def _k1_body(h_ref, wt_ref, wg_ref, bg_ref, hp_ref, gate_ref):
    h = h_ref[...]
    hp_ref[...] = jnp.dot(h, wt_ref[...], preferred_element_type=jnp.float32)
    z = jnp.dot(h, wg_ref[...], preferred_element_type=jnp.float32)
    gate_ref[...] = jax.nn.sigmoid(z + bg_ref[0, 0])


def _k1(h16, W_triple, Wg_t, bg):
    return pl.pallas_call(
        _k1_body,
        out_shape=[jax.ShapeDtypeStruct((16, F), jnp.float32),
                   jax.ShapeDtypeStruct((16, 1), jnp.float32)],
        in_specs=[pl.BlockSpec((16, E), lambda: (0, 0)),
                  pl.BlockSpec((E, F), lambda: (0, 0)),
                  pl.BlockSpec((E, 1), lambda: (0, 0)),
                  pl.BlockSpec(memory_space=pltpu.SMEM)],
        out_specs=[pl.BlockSpec((16, F), lambda: (0, 0)),
                   pl.BlockSpec((16, 1), lambda: (0, 0))],
    )(h16, W_triple, Wg_t, bg)


def _k2_body(hp_ref, tr_ref, lab_ref, out_ref):
    hp = hp_ref[0]
    tr = tr_ref[0]
    logits = lax.dot_general(hp, tr, (((1,), (1,)), ((), ())),
                             preferred_element_type=jnp.float32)
    prob = jax.nn.sigmoid(logits)
    lab = lab_ref[0]
    out_ref[0] = jnp.where(lab == -1, 0.0, prob)


def _k2(hp, triple_repr, labels3):
    return pl.pallas_call(
        _k2_body,
        grid=(B, Mt // BM),
        out_shape=jax.ShapeDtypeStruct((B, L, Mt), jnp.float32),
        in_specs=[pl.BlockSpec((1, L, F), lambda b, j: (b, 0, 0)),
                  pl.BlockSpec((1, BM, F), lambda b, j: (b, j, 0)),
                  pl.BlockSpec((1, 1, BM), lambda b, j: (b, 0, j))],
        out_specs=pl.BlockSpec((1, L, BM), lambda b, j: (b, 0, j)),
    )(hp, triple_repr, labels3)


def _k34_body(tp_hbm, head_hbm, tail_hbm, tlab_hbm, clab_hbm, dist_hbm,
              vm_hbm, mm_hbm,
              cpv_hbm,
              head_v, tail_v, tlab_v, tp_v, clab_v, dist_v, upd_v, ns_v,
              priv_v, acc_v,
              table_v, vm_v, mm_v, cv_v, shared_cp, sem):
    c = lax.axis_index("c")
    s = lax.axis_index("s")
    pair2 = c * 8 + s // 2
    half = s % 2
    b2 = pair2 // 4
    l2 = pair2 % 4
    base = pl.multiple_of(half * HALF, 32)

    @pl.when(s < 8)
    def _():
        pair = c * 8 + s
        b = pair // 4
        l = pair % 4
        d1 = pltpu.async_copy(head_hbm.at[b], head_v, sem)
        d2 = pltpu.async_copy(tail_hbm.at[b], tail_v, sem)
        d3 = pltpu.async_copy(tlab_hbm.at[b], tlab_v, sem)
        d4 = pltpu.async_copy(tp_hbm.at[b, l], tp_v, sem)
        d5 = pltpu.async_copy(clab_hbm.at[b], clab_v, sem)
        d6 = pltpu.async_copy(dist_hbm.at[b], dist_v, sem)
        d1.wait()
        d2.wait()
        d3.wait()
        d4.wait()
        d5.wait()
        d6.wait()
        one = jnp.ones((16,), jnp.float32)
        zero = jnp.zeros((16,), jnp.float32)
        lane = lax.iota(jnp.int32, 16)
        for i in range(Mc // 16):
            sl = pl.ds(i * 16, 16)
            f0 = jnp.where(dist_v[sl] == 0, one, zero)
            acc_v[sl] = f0
            ns_v[sl] = jnp.where(clab_v[sl] == -1, zero, f0)
        for _hop in range(2):
            for i in range(Mt // 16):
                sl = pl.ds(i * 16, 16)
                hs = plsc.load_gather(ns_v, [head_v[sl]])
                upd = hs * GAMMA + tp_v[sl]
                upd_v[sl] = jnp.where(tlab_v[sl] == -1, zero, upd)

            def zbody(i, carry):
                priv_v[pl.ds(i * 16, 16)] = zero
                return carry

            lax.fori_loop(0, 16 * Mc // 16, zbody, 0)

            def scbody(i, carry):
                sl = pl.ds(i * 16, 16)
                addr = lane * Mc + tail_v[sl]
                cur = plsc.load_gather(priv_v, [addr])
                plsc.store_scatter(priv_v, [addr],
                                   jnp.maximum(cur, upd_v[sl]))
                return carry

            lax.fori_loop(0, Mt // 16, scbody, 0)

            def cbody(i, carry):
                sl16 = pl.ds(i * 16, 16)
                m = priv_v[sl16]
                for j in range(1, 16):
                    m = jnp.maximum(m, priv_v[pl.ds(j * Mc + i * 16, 16)])
                o = jnp.where(clab_v[sl16] == -1, zero, m)
                ns_v[sl16] = o
                acc_v[sl16] = acc_v[sl16] + o
                return carry

            lax.fori_loop(0, Mc // 16, cbody, 0)
        mxv = acc_v[pl.ds(0, 16)]
        for i in range(1, Mc // 16):
            mxv = jnp.maximum(mxv, acc_v[pl.ds(i * 16, 16)])
        m = jnp.max(mxv)
        sv = jnp.zeros((16,), jnp.float32)
        for i in range(Mc // 16):
            sl = pl.ds(i * 16, 16)
            e = jnp.exp(acc_v[sl] - m)
            acc_v[sl] = e
            sv = sv + e
        inv_v = one / (zero + jnp.sum(sv))
        for i in range(Mc // 16):
            sl = pl.ds(i * 16, 16)
            acc_v[sl] = acc_v[sl] * inv_v
        pltpu.sync_copy(acc_v, shared_cp.at[s])

    plsc.subcore_barrier()

    pltpu.sync_copy(shared_cp.at[s // 2], table_v)
    zero = jnp.zeros((16,), jnp.float32)

    def inner(k, carry):
        sl = pl.ds(k * 16, 16)
        cpx = plsc.load_gather(table_v, [vm_v[sl]])
        cv_v[sl] = jnp.where(mm_v[sl] == 0, zero, cpx)
        return carry

    d_vm = pltpu.async_copy(vm_hbm.at[b2, pl.ds(base, CH)], vm_v, sem)
    d_mm = pltpu.async_copy(mm_hbm.at[b2, pl.ds(base, CH)], mm_v, sem)
    d_vm.wait()
    d_mm.wait()
    lax.fori_loop(0, NCH, inner, 0)
    pltpu.sync_copy(cv_v, cpv_hbm.at[b2, l2, pl.ds(base, CH)])
    base1 = pl.multiple_of(base + CH, 32)
    d_vm1 = pltpu.async_copy(vm_hbm.at[b2, pl.ds(base1, CH)], vm_v, sem)
    d_mm1 = pltpu.async_copy(mm_hbm.at[b2, pl.ds(base1, CH)], mm_v, sem)
    d_vm1.wait()
    d_mm1.wait()
    lax.fori_loop(0, NCH, inner, 0)
    pltpu.sync_copy(cv_v, cpv_hbm.at[b2, l2, pl.ds(base1, CH)])


def _k34(tp, head_idx, tail_idx, triple_labels, concept_labels, distances,
         vm_p, mm_p):
    mesh = plsc.VectorSubcoreMesh(core_axis_name="c", subcore_axis_name="s")
    f = functools.partial(
        pl.kernel, _k34_body, mesh=mesh,
        compiler_params=pltpu.CompilerParams(needs_layout_passes=False),
        out_type=jax.ShapeDtypeStruct((B, L, VP), jnp.float32),
        scratch_types=[
            pltpu.VMEM((Mt,), jnp.int32),
            pltpu.VMEM((Mt,), jnp.int32),
            pltpu.VMEM((Mt,), jnp.int32),
            pltpu.VMEM((Mt,), jnp.float32),
            pltpu.VMEM((Mc,), jnp.int32),
            pltpu.VMEM((Mc,), jnp.int32),
            pltpu.VMEM((Mt,), jnp.float32),
            pltpu.VMEM((Mc,), jnp.float32),
            pltpu.VMEM((16 * Mc,), jnp.float32),
            pltpu.VMEM((Mc,), jnp.float32),
            pltpu.VMEM((Mc,), jnp.float32),
            pltpu.VMEM((CH,), jnp.int32),
            pltpu.VMEM((CH,), jnp.int32),
            pltpu.VMEM((CH,), jnp.float32),
            pltpu.VMEM_SHARED((8, Mc), jnp.float32),
            pltpu.SemaphoreType.DMA,
        ],
    )()
    return f(tp, head_idx, tail_idx, triple_labels, concept_labels, distances,
             vm_p, mm_p)


def _k5_body(cpvp_ref, lm_ref, gate_ref, probs_ref, cpv_ref, isc_ref):
    cpx = cpvp_ref[0, :, :V]
    lm = lm_ref[0]
    gcol = gate_ref[0, 0].reshape(L, 1)
    p = gcol * cpx + (1.0 - gcol) * lm
    probs_ref[0] = p
    cpv_ref[0] = cpx
    iota = lax.broadcasted_iota(jnp.int32, (L, V), 1)
    mP = jnp.max(p, axis=1, keepdims=True)
    mL = jnp.max(lm, axis=1, keepdims=True)
    iP = jnp.min(jnp.where(p == mP, iota, BIGI), axis=1)
    iL = jnp.min(jnp.where(lm == mL, iota, BIGI), axis=1)
    isc_ref[0, 0] = jnp.where(iP != iL, 1, 0).astype(jnp.int32)


def _k5(cpv_p, lm_probs, gate_bl):
    return pl.pallas_call(
        _k5_body,
        grid=(B,),
        out_shape=[jax.ShapeDtypeStruct((B, L, V), jnp.float32),
                   jax.ShapeDtypeStruct((B, L, V), jnp.float32),
                   jax.ShapeDtypeStruct((B, 1, L), jnp.int32)],
        in_specs=[pl.BlockSpec((1, L, VP), lambda b: (b, 0, 0)),
                  pl.BlockSpec((1, L, V), lambda b: (b, 0, 0)),
                  pl.BlockSpec((1, 1, L), lambda b: (b, 0, 0))],
        out_specs=[pl.BlockSpec((1, L, V), lambda b: (b, 0, 0)),
                   pl.BlockSpec((1, L, V), lambda b: (b, 0, 0)),
                   pl.BlockSpec((1, 1, L), lambda b: (b, 0, 0))],
    )(cpv_p, lm_probs, gate_bl)


def kernel(lm_hidden_states, lm_probs, triple_repr, triple_labels,
           concept_labels, distances, head_idx, tail_idx, vocab_map,
           map_mask, W_triple, W_gate, b_gate):
    h16 = lm_hidden_states.reshape(16, E)
    bg = b_gate.reshape(1, 1)
    h_proj, gate16 = _k1(h16, W_triple, W_gate.T, bg)
    hp = h_proj.reshape(B, L, F)
    labels3 = triple_labels.reshape(B, 1, Mt)
    triple_prob = _k2(hp, triple_repr, labels3)
    vm_p = jnp.pad(vocab_map, ((0, 0), (0, VP - V)))
    mm_p = jnp.pad(map_mask, ((0, 0), (0, VP - V)))
    cpv_p = _k34(triple_prob, head_idx, tail_idx, triple_labels,
                 concept_labels, distances, vm_p, mm_p)
    probs, cpv, isc = _k5(cpv_p, lm_probs, gate16.reshape(B, 1, L))
    gate = gate16.reshape(B, L, 1)
    return probs, gate, cpv, triple_prob, isc.reshape(B, L)
